# baseline (speedup 1.0000x reference)
import functools

import jax
import jax.numpy as jnp
from jax import lax
from jax.experimental import pallas as pl
from jax.experimental.pallas import tpu as pltpu
from jax.experimental.pallas import tpu_sc as plsc

_NC = 2
_NS = 16
_NW = _NC * _NS

_GATHER_DNUMS = lax.GatherDimensionNumbers(
    offset_dims=(), collapsed_slice_dims=(0,), start_index_map=(0,))


def _lane_gather(vec, idx):
    return lax.gather(vec, idx[:, None], _GATHER_DNUMS, slice_sizes=(1,),
                      mode=lax.GatherScatterMode.PROMISE_IN_BOUNDS)


def _fcg_body(rows_per_w, rows_last, ch, nchunk, pc_hbm, ci_hbm, out_hbm,
              pc_v, ci_v, out_v):
    wid = lax.axis_index("s") * _NC + lax.axis_index("c")
    base = wid * rows_per_w
    nrows = jnp.where(wid == _NW - 1, rows_last, rows_per_w)
    last_start = base + nrows - ch

    lane = lax.iota(jnp.int32, 16)
    col = lane & 3
    shl = jnp.minimum(col, 1)
    msk = jnp.minimum(col, 1)
    shr = jnp.maximum(col - 1, 0)
    row_in = lane >> 2

    n_groups = ch // 16

    def chunk_body(i, carry):
        start = jnp.minimum(base + i * ch, last_start)
        pltpu.sync_copy(pc_hbm.at[pl.ds(start * 4, ch * 4)], pc_v)
        pltpu.sync_copy(ci_hbm.at[pl.ds(start, ch)], ci_v)

        def group_body(g, c2):
            civ = ci_v[pl.ds(g * 16, 16)]
            for t in range(4):
                j = g * 4 + t
                pc = pc_v[pl.ds(j * 16, 16)]
                ci = _lane_gather(civ, row_in + 4 * t)
                out_v[pl.ds(j * 16, 16)] = (pc << shl) + ((ci >> shr) & msk)
            return c2

        lax.fori_loop(0, n_groups, group_body, 0)
        pltpu.sync_copy(out_v, out_hbm.at[pl.ds(start * 4, ch * 4)])
        return carry

    lax.fori_loop(0, nchunk, chunk_body, 0)


def kernel(parent_coords, child_indices, parent_features):
    m = parent_coords.shape[0]
    assert m % 16 == 0 and m // _NW >= 16
    rows_per_w = (m // _NW) // 16 * 16
    rows_last = m - (_NW - 1) * rows_per_w
    ch = min(4000, rows_per_w)
    nchunk = -(-rows_last // ch)

    pc_flat = parent_coords.reshape(-1)
    ci = child_indices.astype(jnp.int32)

    body = functools.partial(_fcg_body, rows_per_w, rows_last, ch, nchunk)
    out_flat = pl.kernel(
        body,
        out_type=jax.ShapeDtypeStruct((m * 4,), jnp.int32),
        mesh=plsc.VectorSubcoreMesh(core_axis_name="c", subcore_axis_name="s"),
        scratch_types=[
            pltpu.VMEM((ch * 4,), jnp.int32),
            pltpu.VMEM((ch,), jnp.int32),
            pltpu.VMEM((ch * 4,), jnp.int32),
        ],
    )(pc_flat, ci)
    return out_flat.reshape(m, 4), parent_features


# speedup vs baseline: 6.5915x; 6.5915x over previous
import functools

import jax
import jax.numpy as jnp
from jax import lax
from jax.experimental import pallas as pl
from jax.experimental.pallas import tpu as pltpu
from jax.experimental.pallas import tpu_sc as plsc

_NC = 2
_NS = 16
_NW = _NC * _NS


def _fcg_body(rows_per_w, rows_last, ch, nchunk,
              ci_hbm, c1_hbm, c2_hbm, c3_hbm, o1_hbm, o2_hbm, o3_hbm,
              ci_v, c1_v, c2_v, c3_v, o1_v, o2_v, o3_v):
    wid = lax.axis_index("s") * _NC + lax.axis_index("c")
    base = wid * rows_per_w
    nrows = jnp.where(wid == _NW - 1, rows_last, rows_per_w)
    last_start = base + nrows - ch

    n_vregs = ch // 16

    def chunk_body(i, carry):
        start = jnp.minimum(base + i * ch, last_start)
        pltpu.sync_copy(ci_hbm.at[pl.ds(start, ch)], ci_v)
        pltpu.sync_copy(c1_hbm.at[pl.ds(start, ch)], c1_v)
        pltpu.sync_copy(c2_hbm.at[pl.ds(start, ch)], c2_v)
        pltpu.sync_copy(c3_hbm.at[pl.ds(start, ch)], c3_v)

        def vreg_body(g, c2_):
            s = pl.ds(g * 16, 16)
            civ = ci_v[s]
            o1_v[s] = (c1_v[s] << 1) + (civ & 1)
            o2_v[s] = (c2_v[s] << 1) + ((civ >> 1) & 1)
            o3_v[s] = (c3_v[s] << 1) + ((civ >> 2) & 1)
            return c2_

        lax.fori_loop(0, n_vregs, vreg_body, 0)
        pltpu.sync_copy(o1_v, o1_hbm.at[pl.ds(start, ch)])
        pltpu.sync_copy(o2_v, o2_hbm.at[pl.ds(start, ch)])
        pltpu.sync_copy(o3_v, o3_hbm.at[pl.ds(start, ch)])
        return carry

    lax.fori_loop(0, nchunk, chunk_body, 0)


def kernel(parent_coords, child_indices, parent_features):
    m = parent_coords.shape[0]
    assert m % 16 == 0 and m // _NW >= 16
    rows_per_w = (m // _NW) // 16 * 16
    rows_last = m - (_NW - 1) * rows_per_w
    ch = min(8192, rows_per_w) // 16 * 16
    nchunk = -(-rows_last // ch)

    ci = child_indices.astype(jnp.int32)
    fp = jax.ShapeDtypeStruct((m,), jnp.int32)

    body = functools.partial(_fcg_body, rows_per_w, rows_last, ch, nchunk)
    o1, o2, o3 = pl.kernel(
        body,
        out_type=(fp, fp, fp),
        mesh=plsc.VectorSubcoreMesh(core_axis_name="c", subcore_axis_name="s"),
        scratch_types=[pltpu.VMEM((ch,), jnp.int32) for _ in range(7)],
    )(ci, parent_coords[:, 1], parent_coords[:, 2], parent_coords[:, 3])
    out = jnp.stack([parent_coords[:, 0], o1, o2, o3], axis=1)
    return out, parent_features
